# baseline (speedup 1.0000x reference)
import jax
import jax.numpy as jnp
from jax import lax
from jax.experimental import pallas as pl
from jax.experimental.pallas import tpu as pltpu
from jax.experimental.pallas import tpu_sc as plsc

BATCH = 16384
EMBED_DIM = 64
REG_LAMBDA = 0.0001
NW = 32
BPW = BATCH // NW
L = 16
CHUNK = 16
NCHUNK = BPW // CHUNK
TROWS = 16


def _sc_body(user_ref, pos_ref, neg_ref, utab_ref, itab_ref, out_ref,
             uidx_v, pidx_v, nidx_v, tux_v, tpx_v, tnx_v,
             ut_v, pt_v, nt_v, out_v, sem):
    wid = lax.axis_index("s") * 2 + lax.axis_index("c")
    base = wid * BPW

    pltpu.sync_copy(user_ref.at[pl.ds(base, BPW)], uidx_v)
    pltpu.sync_copy(pos_ref.at[pl.ds(base, BPW)], pidx_v)
    pltpu.sync_copy(neg_ref.at[pl.ds(base, BPW)], nidx_v)

    for c in range(NCHUNK):
        sl = pl.ds(c * CHUNK, L)
        tux_v[c, :] = uidx_v[sl] >> 4
        tpx_v[c, :] = pidx_v[sl] >> 4
        tnx_v[c, :] = nidx_v[sl] >> 4

    zero = jnp.zeros((L,), jnp.float32)
    lane = lax.iota(jnp.int32, L)
    perms = [lax.iota(jnp.int32, L) ^ (1 << k) for k in range(4)]
    dnums = lax.GatherDimensionNumbers(
        offset_dims=(), collapsed_slice_dims=(0,), start_index_map=(0,))

    def _lane_sum(v):
        for p in perms:
            v = v + lax.gather(v, p[:, None], dnums, (1,),
                               mode=lax.GatherScatterMode.PROMISE_IN_BOUNDS)
        return v

    def chunk_body(c, carry):
        acc_ls, acc_sq = carry
        tuv = tux_v[c, :]
        tpv = tpx_v[c, :]
        tnv = tnx_v[c, :]
        descs = []
        for j in range(L):
            descs.append(pltpu.async_copy(utab_ref.at[tuv[j]], ut_v.at[j], sem))
            descs.append(pltpu.async_copy(itab_ref.at[tpv[j]], pt_v.at[j], sem))
            descs.append(pltpu.async_copy(itab_ref.at[tnv[j]], nt_v.at[j], sem))
        for d in descs:
            d.wait()
        uvec = uidx_v[pl.ds(c * CHUNK, L)]
        pvec = pidx_v[pl.ds(c * CHUNK, L)]
        nvec = nidx_v[pl.ds(c * CHUNK, L)]
        diffs = zero
        sq = zero
        for j in range(L):
            ru = uvec[j] & (TROWS - 1)
            rp = pvec[j] & (TROWS - 1)
            rn = nvec[j] & (TROWS - 1)
            us = [ut_v[j, ru, pl.ds(k * L, L)] for k in range(4)]
            ps = [pt_v[j, rp, pl.ds(k * L, L)] for k in range(4)]
            nn = [nt_v[j, rn, pl.ds(k * L, L)] for k in range(4)]
            prod = (us[0] * (ps[0] - nn[0]) + us[1] * (ps[1] - nn[1])
                    + us[2] * (ps[2] - nn[2]) + us[3] * (ps[3] - nn[3]))
            diffs = jnp.where(lane == j, _lane_sum(prod), diffs)
            sq = (sq + us[0] * us[0] + us[1] * us[1] + us[2] * us[2]
                  + us[3] * us[3] + ps[0] * ps[0] + ps[1] * ps[1]
                  + ps[2] * ps[2] + ps[3] * ps[3] + nn[0] * nn[0]
                  + nn[1] * nn[1] + nn[2] * nn[2] + nn[3] * nn[3])
        y = jnp.exp(-jnp.abs(diffs))
        z = y / (y + 2.0)
        z2 = z * z
        poly = 1.0 + z2 * (0.33333333 + z2 * (0.2 + z2 * (0.14285714
                                                          + z2 * 0.11111111)))
        log1py = 2.0 * z * poly
        ls = jnp.minimum(diffs, 0.0) - log1py
        return acc_ls + ls, acc_sq + sq

    acc_ls, acc_sq = lax.fori_loop(0, NCHUNK, chunk_body, (zero, zero))
    out_v[0, :] = acc_ls
    out_v[1, :] = acc_sq
    pltpu.sync_copy(out_v, out_ref.at[wid])


def kernel(user, positive, negative, user_table, item_table):
    utab3 = user_table.reshape(62500, TROWS, EMBED_DIM)
    itab3 = item_table.reshape(62500, TROWS, EMBED_DIM)
    mesh = plsc.VectorSubcoreMesh(core_axis_name="c", subcore_axis_name="s")
    tile_t = pltpu.VMEM((CHUNK, TROWS, EMBED_DIM), jnp.float32)
    partials = pl.kernel(
        _sc_body,
        mesh=mesh,
        out_type=jax.ShapeDtypeStruct((NW, 2, L), jnp.float32),
        scratch_types=[
            pltpu.VMEM((BPW,), jnp.int32),
            pltpu.VMEM((BPW,), jnp.int32),
            pltpu.VMEM((BPW,), jnp.int32),
            pltpu.VMEM((NCHUNK, CHUNK), jnp.int32),
            pltpu.VMEM((NCHUNK, CHUNK), jnp.int32),
            pltpu.VMEM((NCHUNK, CHUNK), jnp.int32),
            tile_t, tile_t, tile_t,
            pltpu.VMEM((2, L), jnp.float32),
            pltpu.SemaphoreType.DMA,
        ],
    )(user, positive, negative, utab3, itab3)
    bpr_loss = -jnp.sum(partials[:, 0, :]) / BATCH
    reg_loss = REG_LAMBDA * jnp.sum(partials[:, 1, :]) / (2.0 * BATCH)
    return (bpr_loss, reg_loss)


# speedup vs baseline: 1.1341x; 1.1341x over previous
import jax
import jax.numpy as jnp
from jax import lax
from jax.experimental import pallas as pl
from jax.experimental.pallas import tpu as pltpu
from jax.experimental.pallas import tpu_sc as plsc

BATCH = 16384
EMBED_DIM = 64
REG_LAMBDA = 0.0001
NW = 32
BPW = BATCH // NW
L = 16
CHUNK = 32
NCHUNK = BPW // CHUNK


def _sc_body(user_ref, pos_ref, neg_ref, utab_ref, itab_ref, out_ref,
             uidx_v, pidx_v, nidx_v, tux_v, tpx_v, tnx_v,
             ut_v, pt_v, nt_v, out_v, sem):
    wid = lax.axis_index("s") * 2 + lax.axis_index("c")
    base = wid * BPW

    pltpu.sync_copy(user_ref.at[pl.ds(base, BPW)], uidx_v)
    pltpu.sync_copy(pos_ref.at[pl.ds(base, BPW)], pidx_v)
    pltpu.sync_copy(neg_ref.at[pl.ds(base, BPW)], nidx_v)

    for c in range(NCHUNK):
        for g in range(CHUNK // L):
            sl = pl.ds(c * CHUNK + g * L, L)
            dst = pl.ds(g * L, L)
            tux_v[c, dst] = uidx_v[sl] >> 3
            tpx_v[c, dst] = pidx_v[sl] >> 3
            tnx_v[c, dst] = nidx_v[sl] >> 3

    zero = jnp.zeros((L,), jnp.float32)
    lane = lax.iota(jnp.int32, L)
    perms = [lax.iota(jnp.int32, L) ^ (1 << k) for k in range(4)]
    dnums = lax.GatherDimensionNumbers(
        offset_dims=(), collapsed_slice_dims=(0,), start_index_map=(0,))

    def _lane_sum(v):
        for p in perms:
            v = v + lax.gather(v, p[:, None], dnums, (1,),
                               mode=lax.GatherScatterMode.PROMISE_IN_BOUNDS)
        return v

    def chunk_body(c, carry):
        acc_ls, acc_sq = carry
        descs = []
        for g in range(CHUNK // L):
            tuv = tux_v[c, pl.ds(g * L, L)]
            tpv = tpx_v[c, pl.ds(g * L, L)]
            tnv = tnx_v[c, pl.ds(g * L, L)]
            for j in range(L):
                jj = g * L + j
                descs.append(pltpu.async_copy(utab_ref.at[tuv[j]], ut_v.at[jj], sem))
                descs.append(pltpu.async_copy(itab_ref.at[tpv[j]], pt_v.at[jj], sem))
                descs.append(pltpu.async_copy(itab_ref.at[tnv[j]], nt_v.at[jj], sem))
        for d in descs:
            d.wait()
        for g in range(CHUNK // L):
            uvec = uidx_v[pl.ds(c * CHUNK + g * L, L)]
            pvec = pidx_v[pl.ds(c * CHUNK + g * L, L)]
            nvec = nidx_v[pl.ds(c * CHUNK + g * L, L)]
            diffs = zero
            sq = zero
            for j in range(L):
                jj = g * L + j
                ru = uvec[j] & 7
                rp = pvec[j] & 7
                rn = nvec[j] & 7
                us = [ut_v[jj, ru, pl.ds(k * L, L)] for k in range(4)]
                ps = [pt_v[jj, rp, pl.ds(k * L, L)] for k in range(4)]
                nn = [nt_v[jj, rn, pl.ds(k * L, L)] for k in range(4)]
                prod = (us[0] * (ps[0] - nn[0]) + us[1] * (ps[1] - nn[1])
                        + us[2] * (ps[2] - nn[2]) + us[3] * (ps[3] - nn[3]))
                diffs = jnp.where(lane == j, _lane_sum(prod), diffs)
                sq = (sq + us[0] * us[0] + us[1] * us[1] + us[2] * us[2]
                      + us[3] * us[3] + ps[0] * ps[0] + ps[1] * ps[1]
                      + ps[2] * ps[2] + ps[3] * ps[3] + nn[0] * nn[0]
                      + nn[1] * nn[1] + nn[2] * nn[2] + nn[3] * nn[3])
            y = jnp.exp(-jnp.abs(diffs))
            z = y / (y + 2.0)
            z2 = z * z
            poly = 1.0 + z2 * (0.33333333 + z2 * (0.2 + z2 * (0.14285714
                                                              + z2 * 0.11111111)))
            log1py = 2.0 * z * poly
            ls = jnp.minimum(diffs, 0.0) - log1py
            acc_ls = acc_ls + ls
            acc_sq = acc_sq + sq
        return acc_ls, acc_sq

    acc_ls, acc_sq = lax.fori_loop(0, NCHUNK, chunk_body, (zero, zero))
    out_v[0, :] = acc_ls
    out_v[1, :] = acc_sq
    pltpu.sync_copy(out_v, out_ref.at[wid])


def kernel(user, positive, negative, user_table, item_table):
    utab3 = user_table.reshape(125000, 8, EMBED_DIM)
    itab3 = item_table.reshape(125000, 8, EMBED_DIM)
    mesh = plsc.VectorSubcoreMesh(core_axis_name="c", subcore_axis_name="s")
    tile_t = pltpu.VMEM((CHUNK, 8, EMBED_DIM), jnp.float32)
    partials = pl.kernel(
        _sc_body,
        mesh=mesh,
        out_type=jax.ShapeDtypeStruct((NW, 2, L), jnp.float32),
        scratch_types=[
            pltpu.VMEM((BPW,), jnp.int32),
            pltpu.VMEM((BPW,), jnp.int32),
            pltpu.VMEM((BPW,), jnp.int32),
            pltpu.VMEM((NCHUNK, CHUNK), jnp.int32),
            pltpu.VMEM((NCHUNK, CHUNK), jnp.int32),
            pltpu.VMEM((NCHUNK, CHUNK), jnp.int32),
            tile_t, tile_t, tile_t,
            pltpu.VMEM((2, L), jnp.float32),
            pltpu.SemaphoreType.DMA,
        ],
    )(user, positive, negative, utab3, itab3)
    bpr_loss = -jnp.sum(partials[:, 0, :]) / BATCH
    reg_loss = REG_LAMBDA * jnp.sum(partials[:, 1, :]) / (2.0 * BATCH)
    return (bpr_loss, reg_loss)
